# baseline (speedup 1.0000x reference)
import functools

import jax
import jax.numpy as jnp
from jax import lax
from jax.experimental import pallas as pl
from jax.experimental.pallas import tpu as pltpu
from jax.experimental.pallas import tpu_sc as plsc

NC = 2
NS = 16
NW = NC * NS
L = 16


def _flatten_kernel(n_rows):
    mesh = plsc.VectorSubcoreMesh(core_axis_name="c", subcore_axis_name="s")
    span = (n_rows // NW) & ~7
    tail = n_rows - NW * span

    CH = 3904

    @functools.partial(
        pl.kernel,
        mesh=mesh,
        compiler_params=pltpu.CompilerParams(
            needs_layout_passes=False, use_tc_tiling_on_sc=False
        ),
        out_type=(jax.ShapeDtypeStruct((n_rows,), jnp.float32),
                  jax.ShapeDtypeStruct((n_rows,), jnp.float32)),
        scratch_types=[
            pltpu.VMEM((CH, 1), jnp.float32),
            pltpu.VMEM((span + tail,), jnp.float32),
            pltpu.SemaphoreType.DMA,
        ],
    )
    def k(ubt_hbm, ibt_hbm, ubf_hbm, ibf_hbm, b2_v, b1_v, sem):
        wid = lax.axis_index("s") * NC + lax.axis_index("c")
        base = wid * span
        lane = lax.broadcasted_iota(jnp.int32, (L,), 0)
        zero = jnp.zeros((L,), jnp.int32)
        for src, dst in ((ubt_hbm, ubf_hbm), (ibt_hbm, ibf_hbm)):
            def work(n, _src=src, _dst=dst):
                for off in range(0, n, CH):
                    c = min(CH, n - off)
                    pltpu.sync_copy(_src.at[pl.ds(base + off, c), :],
                                    b2_v.at[pl.ds(0, c), :])

                    def sq(g, _, _off=off):
                        v = plsc.load_gather(b2_v, [g * L + lane, zero])
                        b1_v[pl.ds(_off + g * L, L)] = v
                        return _

                    lax.fori_loop(0, c // L, sq, None)
                pltpu.sync_copy(b1_v.at[pl.ds(0, n)],
                                _dst.at[pl.ds(base, n)])

            @pl.when(wid < NW - 1)
            def _():
                work(span)

            @pl.when(wid == NW - 1)
            def _():
                work(span + tail)

    return k


def _mf_kernel(b_per_w, n_feats):
    mesh = plsc.VectorSubcoreMesh(core_axis_name="c", subcore_axis_name="s")
    B = b_per_w * NW
    lines_per_w = b_per_w // 128

    @functools.partial(
        pl.kernel,
        mesh=mesh,
        compiler_params=pltpu.CompilerParams(needs_layout_passes=False),
        out_type=jax.ShapeDtypeStruct((B,), jnp.float32),
        scratch_types=[
            pltpu.VMEM((n_feats, b_per_w), jnp.float32),
            pltpu.VMEM((n_feats, b_per_w), jnp.float32),
            pltpu.VMEM((b_per_w,), jnp.float32),
            pltpu.VMEM((b_per_w,), jnp.float32),
            pltpu.VMEM((b_per_w,), jnp.float32),
            pltpu.SemaphoreType.DMA,
        ],
    )
    def k(uvt_hbm, ivt_hbm, ub_hbm, ib_hbm, out_hbm,
          us_v, is_v, ub_v, ib_v, out_v, sem):
        wid = lax.axis_index("s") * NC + lax.axis_index("c")
        base = wid * b_per_w
        wsl = pl.ds(base, b_per_w)

        cps = [
            pltpu.async_copy(uvt_hbm.at[:, wsl], us_v, sem),
            pltpu.async_copy(ivt_hbm.at[:, wsl], is_v, sem),
            pltpu.async_copy(ub_hbm.at[wsl], ub_v, sem),
            pltpu.async_copy(ib_hbm.at[wsl], ib_v, sem),
        ]
        for c in cps:
            c.wait()

        def compute(g, _):
            gsl = pl.ds(g * L, L)
            acc = ub_v[gsl] + ib_v[gsl]
            for f in range(n_feats):
                acc = acc + us_v[f, gsl] * is_v[f, gsl]
            out_v[gsl] = acc
            return _

        lax.fori_loop(0, b_per_w // L, compute, None)
        pltpu.sync_copy(out_v, out_hbm.at[wsl])

    return k


def kernel(user, item, u_bias, i_bias, u_embed, i_embed):
    B = user.shape[0]
    n_feats = u_embed.shape[1]
    b_per_w = B // NW

    uvt = jnp.take(u_embed, user, axis=0).T
    ivt = jnp.take(i_embed, item, axis=0).T

    n_rows = u_bias.shape[0]
    ubf, ibf = _flatten_kernel(n_rows)(u_bias, i_bias)
    ub = jnp.take(ubf, user, axis=0)
    ib = jnp.take(ibf, item, axis=0)

    k = _mf_kernel(b_per_w, n_feats)
    return k(uvt, ivt, ub, ib)


# speedup vs baseline: 12.5386x; 12.5386x over previous
import functools

import jax
import jax.numpy as jnp
from jax import lax
from jax.experimental import pallas as pl
from jax.experimental.pallas import tpu as pltpu
from jax.experimental.pallas import tpu_sc as plsc

NC = 2
NS = 16
NW = NC * NS
L = 16


def _mf_kernel(b_per_w, n_feats):
    mesh = plsc.VectorSubcoreMesh(core_axis_name="c", subcore_axis_name="s")
    B = b_per_w * NW
    lines_per_w = b_per_w // 128

    @functools.partial(
        pl.kernel,
        mesh=mesh,
        compiler_params=pltpu.CompilerParams(needs_layout_passes=False),
        out_type=jax.ShapeDtypeStruct((B,), jnp.float32),
        scratch_types=[
            pltpu.VMEM((n_feats, b_per_w), jnp.float32),
            pltpu.VMEM((n_feats, b_per_w), jnp.float32),
            pltpu.VMEM((b_per_w // 128, 128), jnp.float32),
            pltpu.VMEM((b_per_w // 128, 128), jnp.float32),
            pltpu.VMEM((b_per_w,), jnp.float32),
            pltpu.SemaphoreType.DMA,
        ],
    )
    def k(uvt_hbm, ivt_hbm, ub_hbm, ib_hbm, out_hbm,
          us_v, is_v, ub_v, ib_v, out_v, sem):
        wid = lax.axis_index("s") * NC + lax.axis_index("c")
        base = wid * b_per_w
        wsl = pl.ds(base, b_per_w)

        cps = [
            pltpu.async_copy(uvt_hbm.at[:, wsl], us_v, sem),
            pltpu.async_copy(ivt_hbm.at[:, wsl], is_v, sem),
            pltpu.async_copy(
                ub_hbm.at[pl.ds(wid * lines_per_w, lines_per_w), :],
                ub_v, sem),
            pltpu.async_copy(
                ib_hbm.at[pl.ds(wid * lines_per_w, lines_per_w), :],
                ib_v, sem),
        ]
        for c in cps:
            c.wait()

        def compute(g, _):
            gsl = pl.ds(g * L, L)
            lsl = pl.ds((g * L) % 128, L)
            acc = ub_v[(g * L) // 128, lsl] + ib_v[(g * L) // 128, lsl]
            for f in range(n_feats):
                acc = acc + us_v[f, gsl] * is_v[f, gsl]
            out_v[gsl] = acc
            return _

        lax.fori_loop(0, b_per_w // L, compute, None)
        pltpu.sync_copy(out_v, out_hbm.at[wsl])

    return k


def kernel(user, item, u_bias, i_bias, u_embed, i_embed):
    B = user.shape[0]
    n_feats = u_embed.shape[1]
    b_per_w = B // NW

    uvt = jnp.take(u_embed, user, axis=0).T
    ivt = jnp.take(i_embed, item, axis=0).T
    ub = jnp.take(u_bias, user, axis=0).reshape(B // 128, 128)
    ib = jnp.take(i_bias, item, axis=0).reshape(B // 128, 128)

    k = _mf_kernel(b_per_w, n_feats)
    return k(uvt, ivt, ub, ib)
